# baseline (speedup 1.0000x reference)
import functools

import jax
import jax.numpy as jnp
from jax import lax
from jax.experimental import pallas as pl
from jax.experimental.pallas import tpu as pltpu
from jax.experimental.pallas import tpu_sc as plsc

B, N, K = 8, 2048, 20
BN = B * N
EPS = 1e-5
R = 1024
P = 512
CHUNK = 128

_dot_d = functools.partial(
    lax.dot_general, preferred_element_type=jnp.float32,
    precision=lax.Precision.DEFAULT)
_CN = (((1,), (1,)), ((), ()))
_RC = (((1,), (0,)), ((), ()))


def _topk_body(base_b, xt_ref, xf_ref, idx_ref):
    xt = xt_ref[0]
    xf = xf_ref[0]
    g = _dot_d(xt, xf, _CN)
    nt = jnp.sum(xt * xt, axis=1, keepdims=True)
    nf = jnp.sum(xf * xf, axis=1, keepdims=True)
    pd = 2.0 * g - nt - nf.reshape(1, N)

    base = (pl.program_id(0) + base_b) * N
    iota = lax.broadcasted_iota(jnp.int32, (R, N), 1).astype(jnp.float32)
    big = jnp.float32(float(2 * N))
    neg = jnp.float32(-jnp.inf)
    work = pd
    cols = []
    for _ in range(K):
        m = jnp.max(work, axis=1, keepdims=True)
        masked = jnp.where(work == m, iota, big)
        idx = jnp.min(masked, axis=1, keepdims=True)
        cols.append(idx)
        work = jnp.where(masked == idx, neg, work)
    idxf = jnp.concatenate(cols, axis=1)
    idx_ref[0] = idxf.astype(jnp.int32) + base


def _topk(x, base_b):
    bb, n, c = x.shape
    return pl.pallas_call(
        functools.partial(_topk_body, base_b),
        grid=(bb, n // R),
        in_specs=[
            pl.BlockSpec((1, R, c), lambda i, r: (i, r, 0)),
            pl.BlockSpec((1, n, c), lambda i, r: (i, 0, 0)),
        ],
        out_specs=pl.BlockSpec((1, R, K), lambda i, r: (i, r, 0)),
        out_shape=jax.ShapeDtypeStruct((bb, n, K), jnp.int32),
    )(x, x)


def _make_sc_gather(npts):
    info = plsc.get_sparse_core_info()
    nw = info.num_cores * info.num_subcores
    pts_per_w = npts // nw
    nchunk = pts_per_w // CHUNK
    mesh = plsc.VectorSubcoreMesh(core_axis_name="c", subcore_axis_name="s")

    @functools.partial(
        pl.kernel, mesh=mesh,
        out_type=jax.ShapeDtypeStruct((K, npts, 128), jnp.float32),
        scratch_types=[
            pltpu.VMEM((CHUNK,), jnp.int32),
            pltpu.VMEM((CHUNK, 128), jnp.float32),
            pltpu.SemaphoreType.DMA,
        ],
    )
    def sc_gather(x_hbm, idxk_hbm, out_hbm, idx_v, rows_v, sem):
        wid = lax.axis_index("s") * info.num_cores + lax.axis_index("c")
        wstart = wid * pts_per_w

        def chunk_body(i, _):
            p0 = wstart + i * CHUNK
            for k in range(K):
                pltpu.sync_copy(idxk_hbm.at[k, pl.ds(p0, CHUNK)], idx_v)
                pltpu.async_copy(x_hbm.at[idx_v], rows_v, sem).wait()
                pltpu.sync_copy(rows_v, out_hbm.at[k, pl.ds(p0, CHUNK), :])
            return ()

        lax.fori_loop(0, nchunk, chunk_body, ())

    return sc_gather


_sc_gather_half = _make_sc_gather(BN // 2)


def _conv_body(c, xt_ref, feat_ref, wt_ref, s_ref, b_ref, out_ref):
    xt = xt_ref[:, :c]
    s = s_ref[...]
    bb = b_ref[...]
    co = wt_ref.shape[1]
    neg = jnp.float32(-jnp.inf)
    acc = jnp.full((P, co), neg, jnp.float32)
    for k in range(K):
        fk = feat_ref[k][:, :c]
        fcat = jnp.concatenate([fk - xt, xt], axis=1)
        z = _dot_d(fcat, wt_ref[...], _RC) * s + bb
        acc = jnp.maximum(acc, z)
    out = jax.nn.relu(acc)
    if co < 128:
        out = jnp.concatenate(
            [out, jnp.zeros((P, 128 - co), jnp.float32)], axis=1)
    out_ref[...] = out


def _conv(x2d, feat, c, wt, s, b):
    npts = x2d.shape[0]
    co = wt.shape[1]
    return pl.pallas_call(
        functools.partial(_conv_body, c),
        grid=(npts // P,),
        in_specs=[
            pl.BlockSpec((P, 128), lambda i: (i, 0)),
            pl.BlockSpec((K, P, 128), lambda i: (0, i, 0)),
            pl.BlockSpec((2 * c, co), lambda i: (0, 0)),
            pl.BlockSpec((co,), lambda i: (0,)),
            pl.BlockSpec((co,), lambda i: (0,)),
        ],
        out_specs=pl.BlockSpec((P, 128), lambda i: (i, 0)),
        out_shape=jax.ShapeDtypeStruct((npts, 128), jnp.float32),
    )(x2d, feat, wt, s, b)


def _stage(x2d, c, wt, s, b):
    hb = B // 2
    hn = BN // 2
    xh = [x2d[:hn], x2d[hn:]]
    idxk = []
    for h in range(2):
        idx = _topk(xh[h].reshape(hb, N, 128), h * hb)
        idxk.append(jnp.transpose(idx.reshape(hn, K)))
    feat = [_sc_gather_half(x2d, idxk[h]) for h in range(2)]
    outs = [_conv(xh[h], feat[h], c, wt, s, b) for h in range(2)]
    return jnp.concatenate(outs, axis=0)


def _maxpool_body(x1_ref, x2_ref, x3_ref, x4_ref, out_ref):
    out_ref[0] = jnp.concatenate([
        jnp.max(x1_ref[0, :, :64], axis=0),
        jnp.max(x2_ref[0, :, :64], axis=0),
        jnp.max(x3_ref[0, :, :64], axis=0),
        jnp.max(x4_ref[0], axis=0),
    ], axis=0).reshape(1, 320)


def _maxpool(x1, x2, x3, x4):
    return pl.pallas_call(
        _maxpool_body,
        grid=(B,),
        in_specs=[
            pl.BlockSpec((1, N, 128), lambda i: (i, 0, 0)),
            pl.BlockSpec((1, N, 128), lambda i: (i, 0, 0)),
            pl.BlockSpec((1, N, 128), lambda i: (i, 0, 0)),
            pl.BlockSpec((1, N, 128), lambda i: (i, 0, 0)),
        ],
        out_specs=pl.BlockSpec((1, 1, 320), lambda i: (i, 0, 0)),
        out_shape=jax.ShapeDtypeStruct((B, 1, 320), jnp.float32),
    )(x1, x2, x3, x4)


def _mlp_body(h_ref, l1_ref, l1b_ref, s1_ref, b1_ref, l2_ref, l2b_ref,
              s2_ref, b2_ref, l3_ref, l3b_ref, out_ref):
    h = h_ref[...]
    h = jax.nn.relu((_dot_d(h, l1_ref[...], _RC) + l1b_ref[...]) *
                    s1_ref[...] + b1_ref[...])
    h = jax.nn.relu((_dot_d(h, l2_ref[...], _RC) + l2b_ref[...]) *
                    s2_ref[...] + b2_ref[...])
    out_ref[...] = _dot_d(h, l3_ref[...], _RC) + l3b_ref[...]


def _mlp(h, l1t, l1b, s1, b1, l2t, l2b, s2, b2, l3t, l3b):
    return pl.pallas_call(
        _mlp_body,
        out_shape=jax.ShapeDtypeStruct((B, 128), jnp.float32),
    )(h, l1t, l1b, s1, b1, l2t, l2b, s2, b2, l3t, l3b)


def kernel(x, W1, W2, W3, W4, gc1, bc1, gc2, bc2, gc3, bc3, gc4, bc4,
           L1w, L1b, g1, b1, L2w, L2b, g2, b2, L3w, L3b):
    rsq = jnp.sqrt(jnp.float32(1.0 + EPS))

    w1t = jnp.zeros((16, 64), jnp.float32)
    w1t = w1t.at[0:3].set(W1[:, 0:3].T).at[8:11].set(W1[:, 3:6].T)
    xp = jnp.pad(x, ((0, 0), (0, 0), (0, 125))).reshape(BN, 128)

    x1 = _stage(xp, 8, w1t, gc1 / rsq, bc1)
    x2 = _stage(x1, 64, W2.T, gc2 / rsq, bc2)
    x3 = _stage(x2, 64, W3.T, gc3 / rsq, bc3)
    x4 = _stage(x3, 64, W4.T, gc4 / rsq, bc4)

    h = _maxpool(x1.reshape(B, N, 128), x2.reshape(B, N, 128),
                 x3.reshape(B, N, 128), x4.reshape(B, N, 128)).reshape(B, 320)

    l3t = jnp.pad(L3w.T, ((0, 0), (0, 125)))
    l3b = jnp.pad(L3b, (0, 125))
    out = _mlp(h, L1w.T, L1b, g1 / rsq, b1, L2w.T, L2b, g2 / rsq, b2,
               l3t, l3b)
    return out[:, :3]


# speedup vs baseline: 1.3504x; 1.3504x over previous
import functools

import jax
import jax.numpy as jnp
from jax import lax
from jax.experimental import pallas as pl
from jax.experimental.pallas import tpu as pltpu
from jax.experimental.pallas import tpu_sc as plsc

B, N, K = 8, 2048, 20
BN = B * N
EPS = 1e-5
R = 512
P = 512
CHUNK = 128

_dot_d = functools.partial(
    lax.dot_general, preferred_element_type=jnp.float32,
    precision=lax.Precision.DEFAULT)
_CN = (((1,), (1,)), ((), ()))
_RC = (((1,), (0,)), ((), ()))


def _topk_body(base_b, xt_ref, xf_ref, idx_ref):
    xt = xt_ref[0]
    xf = xf_ref[0]
    g = _dot_d(xt, xf, _CN)
    nt = jnp.sum(xt * xt, axis=1, keepdims=True)
    nf = jnp.sum(xf * xf, axis=1, keepdims=True)
    pd = 2.0 * g - nt - nf.reshape(1, N)

    base = (pl.program_id(0) + base_b) * N
    iota = lax.broadcasted_iota(jnp.int32, (R, N), 1).astype(jnp.float32)
    big = jnp.float32(float(2 * N))
    neg = jnp.float32(-jnp.inf)
    work = pd
    cols = []
    for _ in range(K):
        m = jnp.max(work, axis=1, keepdims=True)
        masked = jnp.where(work == m, iota, big)
        idx = jnp.min(masked, axis=1, keepdims=True)
        cols.append(idx)
        work = jnp.where(masked == idx, neg, work)
    idxf = jnp.concatenate(cols, axis=1)
    idx_ref[0] = idxf.astype(jnp.int32) + base


def _topk(x, base_b):
    bb, n, c = x.shape
    return pl.pallas_call(
        functools.partial(_topk_body, base_b),
        grid=(bb, n // R),
        in_specs=[
            pl.BlockSpec((1, R, c), lambda i, r: (i, r, 0)),
            pl.BlockSpec((1, n, c), lambda i, r: (i, 0, 0)),
        ],
        out_specs=pl.BlockSpec((1, R, K), lambda i, r: (i, r, 0)),
        out_shape=jax.ShapeDtypeStruct((bb, n, K), jnp.int32),
    )(x, x)


def _make_sc_gather(npts):
    info = plsc.get_sparse_core_info()
    nw = info.num_cores * info.num_subcores
    pts_per_w = npts // nw
    nchunk = pts_per_w // CHUNK
    mesh = plsc.VectorSubcoreMesh(core_axis_name="c", subcore_axis_name="s")

    @functools.partial(
        pl.kernel, mesh=mesh,
        out_type=jax.ShapeDtypeStruct((K, npts, 128), jnp.float32),
        scratch_types=[
            pltpu.VMEM((CHUNK,), jnp.int32),
            pltpu.VMEM((CHUNK, 128), jnp.float32),
            pltpu.SemaphoreType.DMA,
        ],
    )
    def sc_gather(x_hbm, idxk_hbm, out_hbm, idx_v, rows_v, sem):
        wid = lax.axis_index("s") * info.num_cores + lax.axis_index("c")
        wstart = wid * pts_per_w

        def chunk_body(i, _):
            p0 = wstart + i * CHUNK
            for k in range(K):
                pltpu.sync_copy(idxk_hbm.at[k, pl.ds(p0, CHUNK)], idx_v)
                pltpu.async_copy(x_hbm.at[idx_v], rows_v, sem).wait()
                pltpu.sync_copy(rows_v, out_hbm.at[k, pl.ds(p0, CHUNK), :])
            return ()

        lax.fori_loop(0, nchunk, chunk_body, ())

    return sc_gather


NSPLIT = 4
_sc_gather_part = _make_sc_gather(BN // NSPLIT)


def _conv_body(c, xt_ref, feat_ref, wt_ref, s_ref, b_ref, out_ref):
    xt = xt_ref[:, :c]
    s = s_ref[...]
    bb = b_ref[...]
    co = wt_ref.shape[1]
    neg = jnp.float32(-jnp.inf)
    acc = jnp.full((P, co), neg, jnp.float32)
    for k in range(K):
        fk = feat_ref[k][:, :c]
        fcat = jnp.concatenate([fk - xt, xt], axis=1)
        z = _dot_d(fcat, wt_ref[...], _RC) * s + bb
        acc = jnp.maximum(acc, z)
    out = jax.nn.relu(acc)
    if co < 128:
        out = jnp.concatenate(
            [out, jnp.zeros((P, 128 - co), jnp.float32)], axis=1)
    out_ref[...] = out


def _conv(x2d, feat, c, wt, s, b):
    npts = x2d.shape[0]
    co = wt.shape[1]
    return pl.pallas_call(
        functools.partial(_conv_body, c),
        grid=(npts // P,),
        in_specs=[
            pl.BlockSpec((P, 128), lambda i: (i, 0)),
            pl.BlockSpec((K, P, 128), lambda i: (0, i, 0)),
            pl.BlockSpec((2 * c, co), lambda i: (0, 0)),
            pl.BlockSpec((co,), lambda i: (0,)),
            pl.BlockSpec((co,), lambda i: (0,)),
        ],
        out_specs=pl.BlockSpec((P, 128), lambda i: (i, 0)),
        out_shape=jax.ShapeDtypeStruct((npts, 128), jnp.float32),
    )(x2d, feat, wt, s, b)


def _stage(x2d, c, wt, s, b):
    hb = B // NSPLIT
    hn = BN // NSPLIT
    xh = [x2d[h * hn:(h + 1) * hn] for h in range(NSPLIT)]
    idxk = []
    for h in range(NSPLIT):
        idx = _topk(xh[h].reshape(hb, N, 128), h * hb)
        idxk.append(jnp.transpose(idx.reshape(hn, K)))
    feat = [_sc_gather_part(x2d, idxk[h]) for h in range(NSPLIT)]
    outs = [_conv(xh[h], feat[h], c, wt, s, b) for h in range(NSPLIT)]
    return jnp.concatenate(outs, axis=0)


def _maxpool_body(x1_ref, x2_ref, x3_ref, x4_ref, out_ref):
    out_ref[0] = jnp.concatenate([
        jnp.max(x1_ref[0, :, :64], axis=0),
        jnp.max(x2_ref[0, :, :64], axis=0),
        jnp.max(x3_ref[0, :, :64], axis=0),
        jnp.max(x4_ref[0], axis=0),
    ], axis=0).reshape(1, 320)


def _maxpool(x1, x2, x3, x4):
    return pl.pallas_call(
        _maxpool_body,
        grid=(B,),
        in_specs=[
            pl.BlockSpec((1, N, 128), lambda i: (i, 0, 0)),
            pl.BlockSpec((1, N, 128), lambda i: (i, 0, 0)),
            pl.BlockSpec((1, N, 128), lambda i: (i, 0, 0)),
            pl.BlockSpec((1, N, 128), lambda i: (i, 0, 0)),
        ],
        out_specs=pl.BlockSpec((1, 1, 320), lambda i: (i, 0, 0)),
        out_shape=jax.ShapeDtypeStruct((B, 1, 320), jnp.float32),
    )(x1, x2, x3, x4)


def _mlp_body(h_ref, l1_ref, l1b_ref, s1_ref, b1_ref, l2_ref, l2b_ref,
              s2_ref, b2_ref, l3_ref, l3b_ref, out_ref):
    h = h_ref[...]
    h = jax.nn.relu((_dot_d(h, l1_ref[...], _RC) + l1b_ref[...]) *
                    s1_ref[...] + b1_ref[...])
    h = jax.nn.relu((_dot_d(h, l2_ref[...], _RC) + l2b_ref[...]) *
                    s2_ref[...] + b2_ref[...])
    out_ref[...] = _dot_d(h, l3_ref[...], _RC) + l3b_ref[...]


def _mlp(h, l1t, l1b, s1, b1, l2t, l2b, s2, b2, l3t, l3b):
    return pl.pallas_call(
        _mlp_body,
        out_shape=jax.ShapeDtypeStruct((B, 128), jnp.float32),
    )(h, l1t, l1b, s1, b1, l2t, l2b, s2, b2, l3t, l3b)


def kernel(x, W1, W2, W3, W4, gc1, bc1, gc2, bc2, gc3, bc3, gc4, bc4,
           L1w, L1b, g1, b1, L2w, L2b, g2, b2, L3w, L3b):
    rsq = jnp.sqrt(jnp.float32(1.0 + EPS))

    w1t = jnp.zeros((16, 64), jnp.float32)
    w1t = w1t.at[0:3].set(W1[:, 0:3].T).at[8:11].set(W1[:, 3:6].T)
    xp = jnp.pad(x, ((0, 0), (0, 0), (0, 125))).reshape(BN, 128)

    x1 = _stage(xp, 8, w1t, gc1 / rsq, bc1)
    x2 = _stage(x1, 64, W2.T, gc2 / rsq, bc2)
    x3 = _stage(x2, 64, W3.T, gc3 / rsq, bc3)
    x4 = _stage(x3, 64, W4.T, gc4 / rsq, bc4)

    h = _maxpool(x1.reshape(B, N, 128), x2.reshape(B, N, 128),
                 x3.reshape(B, N, 128), x4.reshape(B, N, 128)).reshape(B, 320)

    l3t = jnp.pad(L3w.T, ((0, 0), (0, 125)))
    l3b = jnp.pad(L3b, (0, 125))
    out = _mlp(h, L1w.T, L1b, g1 / rsq, b1, L2w.T, L2b, g2 / rsq, b2,
               l3t, l3b)
    return out[:, :3]
